# baseline (speedup 1.0000x reference)
import functools

import jax
import jax.numpy as jnp
import numpy as np
from jax import lax
from jax.experimental import pallas as pl
from jax.experimental.pallas import tpu as pltpu
from jax.experimental.pallas import tpu_sc as plsc

T = 32
N = 8192
NUM_REGIONS = 4
REGION = N // NUM_REGIONS
K_IN = 102
K_OUT = 409
NBLK = 8
BLK = N // NBLK

_TOP = np.uint32(0x80000000)


def _sc_count(m_v, base, width, cand, strict):
    def body(j, cnt):
        off = base + j * 128
        acc = cnt
        for q in range(8):
            v = m_v[pl.ds(off + q * 16, 16)]
            sel = (v > cand) if strict else (v >= cand)
            acc = acc + jnp.where(sel, 1, 0)
        return acc
    cnt16 = lax.fori_loop(0, width // 128, body, jnp.zeros((16,), jnp.int32))
    return jnp.sum(cnt16)


def _sc_count_band(b_v, nvec, cand):
    def body(j, cnt):
        v = b_v[pl.ds(j * 16, 16)]
        return cnt + jnp.where(v >= cand, 1, 0)
    cnt16 = lax.fori_loop(0, nvec, body, jnp.zeros((16,), jnp.int32))
    return jnp.sum(cnt16)


_P1 = 12
_P2 = 32 - _P1


def _sc_topk_row(m_v, o_v, b_v, base, width, k):
    prefix = jnp.uint32(0)
    for bit in range(31, 31 - _P1, -1):
        cand = prefix | jnp.uint32(1 << bit)
        cnt = _sc_count(m_v, base, width, cand, False)
        prefix = jnp.where(cnt >= k, cand, prefix)
    band_top = prefix | jnp.uint32((1 << _P2) - 1)
    c_hi = _sc_count(m_v, base, width, band_top, True)

    p1 = prefix >> _P2

    def cbody(j, off):
        v = m_v[pl.ds(base + j * 16, 16)]
        sel = (v >> _P2) == p1
        plsc.store_compressed(b_v.at[pl.ds(off, 16)], v, mask=sel)
        return off + plsc.all_reduce_population_count(sel)[0]

    n_band = lax.fori_loop(0, width // 16, cbody, jnp.int32(0))
    b_v[pl.ds(n_band, 16)] = jnp.zeros((16,), jnp.uint32)
    nvec = (n_band + 15) >> 4

    k2 = k - c_hi
    for bit in range(_P2 - 1, -1, -1):
        cand = prefix | jnp.uint32(1 << bit)
        cnt = _sc_count_band(b_v, nvec, cand)
        prefix = jnp.where(cnt >= k2, cand, prefix)
    cnt_ge = c_hi + _sc_count_band(b_v, nvec, prefix)

    def simple(_):
        def body(j, c):
            for q in range(4):
                off = base + j * 64 + q * 16
                sel = m_v[pl.ds(off, 16)] >= prefix
                o_v[pl.ds(off, 16)] = jnp.where(sel, 1.0, 0.0).astype(jnp.float32)
            return c
        return lax.fori_loop(0, width // 64, body, jnp.int32(0))

    def with_ties(_):
        cnt_gt = _sc_count(m_v, base, width, prefix, True)
        need = k - cnt_gt

        def body(j, acc):
            off = base + j * 16
            v = m_v[pl.ds(off, 16)]
            gt = v > prefix
            eq = v == prefix
            eqi = jnp.where(eq, 1, 0)
            rank = jnp.cumsum(eqi)
            sel = jnp.logical_or(gt, jnp.logical_and(eq, rank + acc <= need))
            o_v[pl.ds(off, 16)] = jnp.where(sel, 1.0, 0.0).astype(jnp.float32)
            return acc + jnp.sum(eqi)
        return lax.fori_loop(0, width // 16, body, jnp.int32(0))

    lax.cond(cnt_ge == k, simple, with_ties, 0)


def _sc_inmask_body(x_hbm, n_hbm, o_hbm, x_v, n_v, m_v, o_v, b_v):
    c = lax.axis_index("c")
    s = lax.axis_index("s")
    t = c * 16 + s
    pltpu.sync_copy(x_hbm.at[t], x_v)
    pltpu.sync_copy(n_hbm.at[t], n_v)

    ninf = jnp.full((16,), -jnp.inf, jnp.float32)
    pinf = jnp.full((16,), jnp.inf, jnp.float32)

    def mmbody(j, carry):
        mx, mn = carry
        off = j * 64
        for q in range(4):
            v = x_v[pl.ds(off + q * 16, 16)]
            mx = jnp.maximum(mx, v)
            mn = jnp.minimum(mn, v)
        return mx, mn

    mx16, mn16 = lax.fori_loop(0, N // 64, mmbody, (ninf, pinf))
    scale = ((jnp.full((16,), 1e-10, jnp.float32) + jnp.max(mx16))
             - jnp.min(mn16)) / jnp.full((16,), 100.0, jnp.float32)

    def mapbody(j, c):
        for q in range(4):
            off = j * 64 + q * 16
            xn = x_v[pl.ds(off, 16)] + scale * n_v[pl.ds(off, 16)]
            u = plsc.bitcast(xn, jnp.uint32)
            m_v[pl.ds(off, 16)] = jnp.where(u >= _TOP, ~u, u | _TOP)
        return c

    lax.fori_loop(0, N // 64, mapbody, 0)

    for r in range(NUM_REGIONS):
        _sc_topk_row(m_v, o_v, b_v, r * REGION, REGION, K_IN)
        pltpu.sync_copy(o_v.at[pl.ds(r * REGION, REGION)], o_hbm.at[r, t])


def _sc_outmask_body(oh_hbm, n_hbm, o_hbm, x_v, n_v, m_v, o_v, b_v):
    c = lax.axis_index("c")
    s = lax.axis_index("s")
    t = c * 16 + s
    pltpu.sync_copy(oh_hbm.at[t], x_v)
    pltpu.sync_copy(n_hbm.at[t], n_v)

    pinf = jnp.full((16,), jnp.inf, jnp.float32)

    def mnbody(j, mn):
        off = j * 64
        for q in range(4):
            mn = jnp.minimum(mn, x_v[pl.ds(off + q * 16, 16)])
        return mn

    mn16 = lax.fori_loop(0, N // 64, mnbody, pinf)
    scale = jnp.abs((jnp.full((16,), 0.0, jnp.float32) + jnp.min(mn16))
                    / jnp.full((16,), 10.0, jnp.float32))

    def mapbody(j, c):
        for q in range(4):
            off = j * 64 + q * 16
            xn = x_v[pl.ds(off, 16)] + scale * n_v[pl.ds(off, 16)]
            u = plsc.bitcast(xn, jnp.uint32)
            m_v[pl.ds(off, 16)] = jnp.where(u >= _TOP, ~u, u | _TOP)
        return c

    lax.fori_loop(0, N // 64, mapbody, 0)
    _sc_topk_row(m_v, o_v, b_v, 0, N, K_OUT)
    pltpu.sync_copy(o_v, o_hbm.at[t])


_sc_mesh = plsc.VectorSubcoreMesh(core_axis_name="c", subcore_axis_name="s")

_sc_inmask = functools.partial(
    pl.kernel,
    mesh=_sc_mesh,
    compiler_params=pltpu.CompilerParams(needs_layout_passes=False),
    out_type=jax.ShapeDtypeStruct((NUM_REGIONS, T, REGION), jnp.float32),
    scratch_types=[
        pltpu.VMEM((N,), jnp.float32),
        pltpu.VMEM((N,), jnp.float32),
        pltpu.VMEM((N,), jnp.uint32),
        pltpu.VMEM((N,), jnp.float32),
        pltpu.VMEM((REGION + 16,), jnp.uint32),
    ],
)(_sc_inmask_body)

_sc_outmask = functools.partial(
    pl.kernel,
    mesh=_sc_mesh,
    compiler_params=pltpu.CompilerParams(needs_layout_passes=False),
    out_type=jax.ShapeDtypeStruct((T, N), jnp.float32),
    scratch_types=[
        pltpu.VMEM((N,), jnp.float32),
        pltpu.VMEM((N,), jnp.float32),
        pltpu.VMEM((N,), jnp.uint32),
        pltpu.VMEM((N,), jnp.float32),
        pltpu.VMEM((N + 16,), jnp.uint32),
    ],
)(_sc_outmask_body)


def _mm_body(m3_ref, w_ref, o_ref, oh3_ref):
    r = pl.program_id(0)
    i = pl.program_id(1)
    p = lax.dot_general(
        m3_ref[0], w_ref[...],
        dimension_numbers=(((1,), (1,)), ((), ())),
        preferred_element_type=jnp.float32,
        precision=lax.Precision.DEFAULT)

    @pl.when(r == 0)
    def _():
        oh3_ref[i] = p

    @pl.when(r > 0)
    def _():
        oh3_ref[i] = oh3_ref[i] + p

    @pl.when(jnp.logical_and(r == NUM_REGIONS - 1, i == NBLK - 1))
    def _():
        o_ref[...] = jnp.concatenate([oh3_ref[j] for j in range(NBLK)], axis=1)


def _matmul(mask3, out_in):
    return pl.pallas_call(
        _mm_body,
        grid=(NUM_REGIONS, NBLK),
        in_specs=[
            pl.BlockSpec((1, T, REGION), lambda r, i: (r, 0, 0)),
            pl.BlockSpec((BLK, REGION), lambda r, i: (i, r)),
        ],
        out_specs=pl.BlockSpec((T, N), lambda r, i: (0, 0)),
        out_shape=jax.ShapeDtypeStruct((T, N), jnp.float32),
        scratch_shapes=[pltpu.VMEM((NBLK, T, BLK), jnp.float32)],
    )(mask3, out_in)


def kernel(input, out_in, test):
    del test
    base = jax.random.key(42)
    keys = jax.vmap(lambda i: jax.random.fold_in(base, i))(jnp.arange(2 * T))
    noise = jax.vmap(
        lambda k: jax.random.normal(k, (N,), dtype=jnp.float32))(keys)
    noise_in = noise[0::2]
    noise_out = noise[1::2]
    mask3 = _sc_inmask(input, noise_in)
    oh = _matmul(mask3, out_in)
    return _sc_outmask(oh, noise_out)


# speedup vs baseline: 1.0033x; 1.0033x over previous
import functools

import jax
import jax.numpy as jnp
import numpy as np
from jax import lax
from jax.experimental import pallas as pl
from jax.experimental.pallas import tpu as pltpu
from jax.experimental.pallas import tpu_sc as plsc

T = 32
N = 8192
NUM_REGIONS = 4
REGION = N // NUM_REGIONS
K_IN = 102
K_OUT = 409
NBLK = 8
BLK = N // NBLK

_TOP = np.uint32(0x80000000)


def _sc_count(m_v, base, width, cand, strict):
    def body(j, cnt):
        off = base + j * 128
        acc = cnt
        for q in range(8):
            v = m_v[pl.ds(off + q * 16, 16)]
            sel = (v > cand) if strict else (v >= cand)
            acc = acc + plsc.all_reduce_population_count(sel)
        return acc
    cnt16 = lax.fori_loop(0, width // 128, body, jnp.zeros((16,), jnp.int32))
    return cnt16[0]


def _sc_count_band(b_v, nvec, cand):
    def body(j, cnt):
        v = b_v[pl.ds(j * 16, 16)]
        return cnt + jnp.where(v >= cand, 1, 0)
    cnt16 = lax.fori_loop(0, nvec, body, jnp.zeros((16,), jnp.int32))
    return jnp.sum(cnt16)


_P1 = 12
_P2 = 32 - _P1


def _sc_topk_row(m_v, o_v, b_v, base, width, k):
    prefix = jnp.uint32(0)
    for bit in range(31, 31 - _P1, -1):
        cand = prefix | jnp.uint32(1 << bit)
        cnt = _sc_count(m_v, base, width, cand, False)
        prefix = jnp.where(cnt >= k, cand, prefix)
    band_top = prefix | jnp.uint32((1 << _P2) - 1)
    c_hi = _sc_count(m_v, base, width, band_top, True)

    p1 = prefix >> _P2

    def cbody(j, off):
        v = m_v[pl.ds(base + j * 16, 16)]
        sel = (v >> _P2) == p1
        plsc.store_compressed(b_v.at[pl.ds(off, 16)], v, mask=sel)
        return off + plsc.all_reduce_population_count(sel)[0]

    n_band = lax.fori_loop(0, width // 16, cbody, jnp.int32(0))
    b_v[pl.ds(n_band, 16)] = jnp.zeros((16,), jnp.uint32)
    nvec = (n_band + 15) >> 4

    k2 = k - c_hi
    for bit in range(_P2 - 1, -1, -1):
        cand = prefix | jnp.uint32(1 << bit)
        cnt = _sc_count_band(b_v, nvec, cand)
        prefix = jnp.where(cnt >= k2, cand, prefix)
    cnt_ge = c_hi + _sc_count_band(b_v, nvec, prefix)

    def simple(_):
        def body(j, c):
            for q in range(4):
                off = base + j * 64 + q * 16
                sel = m_v[pl.ds(off, 16)] >= prefix
                o_v[pl.ds(off, 16)] = jnp.where(sel, 1.0, 0.0).astype(jnp.float32)
            return c
        return lax.fori_loop(0, width // 64, body, jnp.int32(0))

    def with_ties(_):
        cnt_gt = _sc_count(m_v, base, width, prefix, True)
        need = k - cnt_gt

        def body(j, acc):
            off = base + j * 16
            v = m_v[pl.ds(off, 16)]
            gt = v > prefix
            eq = v == prefix
            eqi = jnp.where(eq, 1, 0)
            rank = jnp.cumsum(eqi)
            sel = jnp.logical_or(gt, jnp.logical_and(eq, rank + acc <= need))
            o_v[pl.ds(off, 16)] = jnp.where(sel, 1.0, 0.0).astype(jnp.float32)
            return acc + jnp.sum(eqi)
        return lax.fori_loop(0, width // 16, body, jnp.int32(0))

    lax.cond(cnt_ge == k, simple, with_ties, 0)


def _sc_inmask_body(x_hbm, n_hbm, o_hbm, x_v, n_v, m_v, o_v, b_v):
    c = lax.axis_index("c")
    s = lax.axis_index("s")
    t = c * 16 + s
    pltpu.sync_copy(x_hbm.at[t], x_v)
    pltpu.sync_copy(n_hbm.at[t], n_v)

    ninf = jnp.full((16,), -jnp.inf, jnp.float32)
    pinf = jnp.full((16,), jnp.inf, jnp.float32)

    def mmbody(j, carry):
        mx, mn = carry
        off = j * 64
        for q in range(4):
            v = x_v[pl.ds(off + q * 16, 16)]
            mx = jnp.maximum(mx, v)
            mn = jnp.minimum(mn, v)
        return mx, mn

    mx16, mn16 = lax.fori_loop(0, N // 64, mmbody, (ninf, pinf))
    scale = ((jnp.full((16,), 1e-10, jnp.float32) + jnp.max(mx16))
             - jnp.min(mn16)) / jnp.full((16,), 100.0, jnp.float32)

    def mapbody(j, c):
        for q in range(4):
            off = j * 64 + q * 16
            xn = x_v[pl.ds(off, 16)] + scale * n_v[pl.ds(off, 16)]
            u = plsc.bitcast(xn, jnp.uint32)
            m_v[pl.ds(off, 16)] = jnp.where(u >= _TOP, ~u, u | _TOP)
        return c

    lax.fori_loop(0, N // 64, mapbody, 0)

    for r in range(NUM_REGIONS):
        _sc_topk_row(m_v, o_v, b_v, r * REGION, REGION, K_IN)
        pltpu.sync_copy(o_v.at[pl.ds(r * REGION, REGION)], o_hbm.at[r, t])


def _sc_outmask_body(oh_hbm, n_hbm, o_hbm, x_v, n_v, m_v, o_v, b_v):
    c = lax.axis_index("c")
    s = lax.axis_index("s")
    t = c * 16 + s
    pltpu.sync_copy(oh_hbm.at[t], x_v)
    pltpu.sync_copy(n_hbm.at[t], n_v)

    pinf = jnp.full((16,), jnp.inf, jnp.float32)

    def mnbody(j, mn):
        off = j * 64
        for q in range(4):
            mn = jnp.minimum(mn, x_v[pl.ds(off + q * 16, 16)])
        return mn

    mn16 = lax.fori_loop(0, N // 64, mnbody, pinf)
    scale = jnp.abs((jnp.full((16,), 0.0, jnp.float32) + jnp.min(mn16))
                    / jnp.full((16,), 10.0, jnp.float32))

    def mapbody(j, c):
        for q in range(4):
            off = j * 64 + q * 16
            xn = x_v[pl.ds(off, 16)] + scale * n_v[pl.ds(off, 16)]
            u = plsc.bitcast(xn, jnp.uint32)
            m_v[pl.ds(off, 16)] = jnp.where(u >= _TOP, ~u, u | _TOP)
        return c

    lax.fori_loop(0, N // 64, mapbody, 0)
    _sc_topk_row(m_v, o_v, b_v, 0, N, K_OUT)
    pltpu.sync_copy(o_v, o_hbm.at[t])


_sc_mesh = plsc.VectorSubcoreMesh(core_axis_name="c", subcore_axis_name="s")

_sc_inmask = functools.partial(
    pl.kernel,
    mesh=_sc_mesh,
    compiler_params=pltpu.CompilerParams(needs_layout_passes=False),
    out_type=jax.ShapeDtypeStruct((NUM_REGIONS, T, REGION), jnp.float32),
    scratch_types=[
        pltpu.VMEM((N,), jnp.float32),
        pltpu.VMEM((N,), jnp.float32),
        pltpu.VMEM((N,), jnp.uint32),
        pltpu.VMEM((N,), jnp.float32),
        pltpu.VMEM((REGION + 16,), jnp.uint32),
    ],
)(_sc_inmask_body)

_sc_outmask = functools.partial(
    pl.kernel,
    mesh=_sc_mesh,
    compiler_params=pltpu.CompilerParams(needs_layout_passes=False),
    out_type=jax.ShapeDtypeStruct((T, N), jnp.float32),
    scratch_types=[
        pltpu.VMEM((N,), jnp.float32),
        pltpu.VMEM((N,), jnp.float32),
        pltpu.VMEM((N,), jnp.uint32),
        pltpu.VMEM((N,), jnp.float32),
        pltpu.VMEM((N + 16,), jnp.uint32),
    ],
)(_sc_outmask_body)


def _mm_body(m3_ref, w_ref, o_ref, oh3_ref):
    r = pl.program_id(0)
    i = pl.program_id(1)
    p = lax.dot_general(
        m3_ref[0], w_ref[...],
        dimension_numbers=(((1,), (1,)), ((), ())),
        preferred_element_type=jnp.float32,
        precision=lax.Precision.DEFAULT)

    @pl.when(r == 0)
    def _():
        oh3_ref[i] = p

    @pl.when(r > 0)
    def _():
        oh3_ref[i] = oh3_ref[i] + p

    @pl.when(jnp.logical_and(r == NUM_REGIONS - 1, i == NBLK - 1))
    def _():
        o_ref[...] = jnp.concatenate([oh3_ref[j] for j in range(NBLK)], axis=1)


def _matmul(mask3, out_in):
    return pl.pallas_call(
        _mm_body,
        grid=(NUM_REGIONS, NBLK),
        in_specs=[
            pl.BlockSpec((1, T, REGION), lambda r, i: (r, 0, 0)),
            pl.BlockSpec((BLK, REGION), lambda r, i: (i, r)),
        ],
        out_specs=pl.BlockSpec((T, N), lambda r, i: (0, 0)),
        out_shape=jax.ShapeDtypeStruct((T, N), jnp.float32),
        scratch_shapes=[pltpu.VMEM((NBLK, T, BLK), jnp.float32)],
    )(mask3, out_in)


def kernel(input, out_in, test):
    del test
    base = jax.random.key(42)
    keys = jax.vmap(lambda i: jax.random.fold_in(base, i))(jnp.arange(2 * T))
    noise = jax.vmap(
        lambda k: jax.random.normal(k, (N,), dtype=jnp.float32))(keys)
    noise_in = noise[0::2]
    noise_out = noise[1::2]
    mask3 = _sc_inmask(input, noise_in)
    oh = _matmul(mask3, out_in)
    return _sc_outmask(oh, noise_out)
